# baseline (speedup 1.0000x reference)
import functools

import jax
import jax.numpy as jnp
from jax import lax
from jax.experimental import pallas as pl
from jax.experimental.pallas import tpu as pltpu
from jax.experimental.pallas import tpu_sc as plsc

B = 16384
NF = 16
NCOIL = 1000000
NREC = 100000
_NC = 2
_NS = 16
_NW = _NC * _NS
_SLAB = 2816
_BINCAP = 4096
_IDXCH = 2048
_P2 = B // _NS

_COIL_FULL = (NCOIL // 128) * 128
_REC_FULL = (NREC // 128) * 128


def _i16(v):
    return jnp.full((16,), v, jnp.int32)


def _sc_table_body(ntail, nfull, n_total, nchunk,
                   idx_hbm, tabT, tail_hbm,
                   out0T, out1T,
                   idx_v, mi_v, mj_v, bi_v, bj_v, slab_v, tail_v,
                   zflat_v, eT_v, rowflat_v, idxflat_v,
                   sp_acc):
    c = lax.axis_index("c")
    s = lax.axis_index("s")
    wid = s * _NC + c
    iota = lax.iota(jnp.int32, 16)

    def zrow(r, _):
        zflat_v[pl.ds(r * 16, 16)] = jnp.zeros((16,), jnp.int32)
        return 0
    lax.fori_loop(0, (_P2 * NF // 2) // 16, zrow, 0)
    pltpu.sync_copy(zflat_v,
                    sp_acc.at[pl.ds(s * (_P2 * NF // 2), _P2 * NF // 2)])
    plsc.subcore_barrier()

    full_tiles = nfull // 128
    base_t, extra_t = full_tiles // _NW, full_tiles % _NW
    lo = (wid * base_t + jnp.minimum(wid, extra_t)) * 128
    cnt = jnp.where(wid < extra_t, base_t + 1, base_t) * 128
    hi = lo + cnt
    hi_scan = jnp.where(wid == _NW - 1, jnp.int32(n_total), hi)
    @pl.when(wid == _NW - 1)
    def _():
        pltpu.sync_copy(tabT.at[:, pl.ds(jnp.int32(nfull), ntail)], tail_v)

    def scan_chunk(ic, n):
        pltpu.sync_copy(idx_hbm.at[pl.ds(ic * _IDXCH, _IDXCH)], idx_v)
        def scan_g(g, n):
            idx16 = idx_v[pl.ds(g * 16, 16)]
            m = jnp.logical_and(idx16 >= lo, idx16 < hi_scan)
            plsc.store_compressed(mi_v.at[pl.ds(n, 16)], idx16, mask=m)
            j16 = iota + (ic * _IDXCH + g * 16)
            plsc.store_compressed(mj_v.at[pl.ds(n, 16)], j16, mask=m)
            return n + jnp.max(plsc.all_reduce_population_count(m))
        return lax.fori_loop(0, _IDXCH // 16, scan_g, n, unroll=4)
    n = lax.fori_loop(0, B // _IDXCH, scan_chunk, jnp.int32(0))
    ng = (n + 15) // 16

    def extract_bin(slab_ref, nb, clampst, maxrel):
        def grp(g, _):
            k = g * 16
            rm = bi_v[pl.ds(k, 16)]
            jm = bj_v[pl.ds(k, 16)]
            valid = (iota + k) < nb
            rel = jnp.clip(rm - clampst, 0, jnp.int32(maxrel - 1))
            jbase = jnp.where(valid, jm * (NF // 2),
                              jnp.int32(B * NF // 2) + s)
            for fp in range(NF // 2):
                v0 = plsc.load_gather(slab_ref, [_i16(2 * fp), rel])
                v1 = plsc.load_gather(slab_ref, [_i16(2 * fp + 1), rel])
                w = plsc.pack(v0, v1, format=plsc.PackFormat.INTERLEAVED)
                wi = plsc.bitcast(w, jnp.int32)
                plsc.store_scatter(rowflat_v, [iota * (NF // 2) + fp], wi)
                plsc.store_scatter(idxflat_v, [iota * (NF // 2) + fp],
                                   jbase + fp)
            pltpu.sync_copy(rowflat_v, sp_acc.at[idxflat_v])
            return 0
        lax.fori_loop(0, (nb + 15) // 16, grp, 0)

    def compact_chunk(cov_lo, cov_hi):
        def cg(g, nb):
            rm = mi_v[pl.ds(g * 16, 16)]
            jm = mj_v[pl.ds(g * 16, 16)]
            valid = (iota + g * 16) < n
            m = valid & (rm >= cov_lo) & (rm < cov_hi)
            nbs = jnp.minimum(nb, jnp.int32(_BINCAP - 16))
            plsc.store_compressed(bi_v.at[pl.ds(nbs, 16)], rm, mask=m)
            plsc.store_compressed(bj_v.at[pl.ds(nbs, 16)], jm, mask=m)
            return nbs + jnp.max(plsc.all_reduce_population_count(m))
        return lax.fori_loop(0, ng, cg, jnp.int32(0))

    for ch in range(nchunk):
        cov_lo = lo + ch * _SLAB
        cov_hi = jnp.minimum(cov_lo + _SLAB, hi_scan)
        @pl.when(cov_lo < hi_scan)
        def _():
            clampst = jnp.minimum(cov_lo, jnp.int32(nfull - _SLAB))
            pltpu.sync_copy(tabT.at[:, pl.ds(clampst, _SLAB)], slab_v)
            nb = compact_chunk(jnp.maximum(cov_lo, clampst),
                               jnp.minimum(cov_hi, clampst + _SLAB))
            extract_bin(slab_v, nb, clampst, _SLAB)

    @pl.when(wid == _NW - 1)
    def _():
        nb = compact_chunk(jnp.int32(nfull), jnp.int32(n_total))
        extract_bin(tail_v, nb, jnp.int32(nfull), ntail)

    plsc.subcore_barrier()

    pltpu.sync_copy(sp_acc.at[pl.ds(s * (_P2 * NF // 2), _P2 * NF // 2)],
                    zflat_v)
    def tg(g, _):
        j0 = g * 16
        base = (iota + j0) * (NF // 2)
        for fp in range(NF // 2):
            w = plsc.load_gather(zflat_v, [base + fp])
            wb = plsc.bitcast(w, jnp.bfloat16)
            a, b = plsc.unpack(wb, format=plsc.PackFormat.INTERLEAVED)
            eT_v[2 * fp, pl.ds(j0, 16)] = a
            eT_v[2 * fp + 1, pl.ds(j0, 16)] = b
        return 0
    lax.fori_loop(0, _P2 // 16, tg, 0)
    @pl.when(c == 0)
    def _():
        pltpu.sync_copy(eT_v, out0T.at[:, pl.ds(s * _P2, _P2)])
    @pl.when(c == 1)
    def _():
        pltpu.sync_copy(eT_v, out1T.at[:, pl.ds(s * _P2, _P2)])


@functools.cache
def _sc_table_kernel(ntail, nfull, n_total, nchunk):
    f32, i32 = jnp.float32, jnp.int32
    outs = [
        jax.ShapeDtypeStruct((NF, B), f32),
        jax.ShapeDtypeStruct((NF, B), f32),
    ]
    kern = functools.partial(
        pl.kernel,
        mesh=plsc.VectorSubcoreMesh(core_axis_name="c", subcore_axis_name="s"),
        compiler_params=pltpu.CompilerParams(needs_layout_passes=False),
        out_type=outs,
        scratch_types=[
            pltpu.VMEM((_IDXCH,), i32),
            pltpu.VMEM((B,), i32),
            pltpu.VMEM((B,), i32),
            pltpu.VMEM((_BINCAP,), i32),
            pltpu.VMEM((_BINCAP,), i32),
            pltpu.VMEM((NF, _SLAB), f32),
            pltpu.VMEM((NF, ntail), f32),
            pltpu.VMEM((_P2 * NF // 2,), i32),
            pltpu.VMEM((NF, _P2), f32),
            pltpu.VMEM((128,), i32),
            pltpu.VMEM((128,), i32),
            pltpu.VMEM_SHARED((B * NF // 2 + 16,), i32),
        ],
    )

    def body(idx_hbm, tabT, *rest):
        return _sc_table_body(ntail, nfull, n_total, nchunk,
                              idx_hbm, tabT, None, *rest)

    return kern(body)


def _sc_bias_body(ci_hbm, ri_hbm, cb_hbm, rb_hbm, bs_out,
                  bidx_v, cb_v, rb_v, bsem):
    c = lax.axis_index("c")
    s = lax.axis_index("s")
    wid = s * _NC + c
    b0 = wid * 512
    pltpu.sync_copy(ci_hbm.at[pl.ds(b0, 512)], bidx_v)
    pltpu.async_copy(cb_hbm.at[bidx_v], cb_v, bsem).wait()
    pltpu.sync_copy(ri_hbm.at[pl.ds(b0, 512)], bidx_v)
    pltpu.async_copy(rb_hbm.at[bidx_v], rb_v, bsem).wait()
    def bsum(g, _):
        sl = pl.ds(g * 16, 16)
        cb_v[sl] = cb_v[sl] + rb_v[sl]
        return 0
    lax.fori_loop(0, 512 // 16, bsum, 0, unroll=4)
    pltpu.sync_copy(cb_v, bs_out.at[pl.ds(b0, 512)])


@functools.cache
def _sc_bias_kernel():
    f32, i32 = jnp.float32, jnp.int32
    return functools.partial(
        pl.kernel,
        mesh=plsc.VectorSubcoreMesh(core_axis_name="c", subcore_axis_name="s"),
        compiler_params=pltpu.CompilerParams(needs_layout_passes=False),
        out_type=[jax.ShapeDtypeStruct((B,), f32)],
        scratch_types=[
            pltpu.VMEM((512,), i32),
            pltpu.VMEM((512,), f32),
            pltpu.VMEM((512,), f32),
            pltpu.SemaphoreType.DMA,
        ],
    )(_sc_bias_body)


_BLK = 2048


def _tc_mlp_body(cfT, rfT, ce0, ce1, re0, re1, bs,
                 w1aT, w1bT, b1, w2T, b2,
                 ew1hT, ew1fT, eb1, ew2T, eb2, woT, bo, out):
    hT = jnp.maximum(
        jnp.dot(w1aT[...], cfT[...], preferred_element_type=jnp.float32)
        + jnp.dot(w1bT[...], rfT[...], preferred_element_type=jnp.float32)
        + b1[...], 0.0)
    h2T = jnp.maximum(
        jnp.dot(w2T[...], hT, preferred_element_type=jnp.float32) + b2[...],
        0.0)
    fmT = (ce0[...] + ce1[...]) * (re0[...] + re1[...])
    e1T = jnp.maximum(
        jnp.dot(ew1hT[...], h2T, preferred_element_type=jnp.float32)
        + jnp.dot(ew1fT[...], fmT, preferred_element_type=jnp.float32)
        + eb1[...], 0.0)
    e2T = jnp.maximum(
        jnp.dot(ew2T[...], e1T, preferred_element_type=jnp.float32)
        + eb2[...], 0.0)
    eoT = jnp.dot(woT[...], e2T, preferred_element_type=jnp.float32) + bo[...]
    out[...] = bs[...] + eoT


def _full(shape):
    return pl.BlockSpec(shape, lambda i: (0, 0))


def _tc_mlp(cfT, rfT, ce0, ce1, re0, re1, bs,
            w1aT, w1bT, b1, w2T, b2,
            ew1hT, ew1fT, eb1, ew2T, eb2, woT, bo, interpret=False):
    grid = (B // _BLK,)
    colblk = lambda h: pl.BlockSpec((h, _BLK), lambda i: (0, i))
    return pl.pallas_call(
        _tc_mlp_body,
        grid=grid,
        in_specs=[
            colblk(64), colblk(64),
            colblk(NF), colblk(NF), colblk(NF), colblk(NF), colblk(1),
            _full((16, 64)), _full((16, 64)), _full((16, 1)),
            _full((8, 16)), _full((8, 1)),
            _full((8, 8)), _full((8, NF)), _full((8, 1)),
            _full((4, 8)), _full((4, 1)),
            _full((1, 4)), _full((1, 1)),
        ],
        out_specs=pl.BlockSpec((1, _BLK), lambda i: (0, i)),
        out_shape=jax.ShapeDtypeStruct((1, B), jnp.float32),
        interpret=interpret,
    )(cfT, rfT, ce0, ce1, re0, re1, bs,
      w1aT, w1bT, b1, w2T, b2,
      ew1hT, ew1fT, eb1, ew2T, eb2, woT, bo)


def kernel(coil_indices, recipe_indices, coil_features, recipe_features,
           coil_emb, recipe_emb, coil_bias, recipe_bias,
           mlp_W1, mlp_b1, mlp_W2, mlp_b2,
           ens_W1, ens_b1, ens_W2, ens_b2, ens_Wo, ens_bo):
    ci = coil_indices.astype(jnp.int32)
    ri = recipe_indices.astype(jnp.int32)
    ce0, ce1 = _sc_table_kernel(NCOIL - _COIL_FULL, _COIL_FULL,
                                NCOIL, 12)(
        ci, coil_emb.T)
    re0, re1 = _sc_table_kernel(NREC - _REC_FULL, _REC_FULL,
                                NREC, 2)(
        ri, recipe_emb.T)
    (bs,) = _sc_bias_kernel()(
        ci, ri, coil_bias.reshape(-1), recipe_bias.reshape(-1))
    w1T = mlp_W1.T
    pred = _tc_mlp(
        coil_features.T, recipe_features.T, ce0, ce1, re0, re1,
        bs.reshape(1, B),
        w1T[:, :64], w1T[:, 64:], mlp_b1.reshape(16, 1),
        mlp_W2.T, mlp_b2.reshape(8, 1),
        ens_W1.T[:, :8], ens_W1.T[:, 8:], ens_b1.reshape(8, 1),
        ens_W2.T, ens_b2.reshape(4, 1),
        ens_Wo.T, ens_bo.reshape(1, 1))
    return pred[0]


# speedup vs baseline: 1.0132x; 1.0132x over previous
import functools

import jax
import jax.numpy as jnp
from jax import lax
from jax.experimental import pallas as pl
from jax.experimental.pallas import tpu as pltpu
from jax.experimental.pallas import tpu_sc as plsc

B = 16384
NF = 16
NCOIL = 1000000
NREC = 100000
_NC = 2
_NS = 16
_NW = _NC * _NS
_SLAB = 2816
_BINCAP = 4096
_IDXCH = 2048
_P2 = B // _NS

_COIL_FULL = (NCOIL // 128) * 128
_REC_FULL = (NREC // 128) * 128


def _i16(v):
    return jnp.full((16,), v, jnp.int32)


def _sc_table_body(ntail, nfull, n_total, nchunk,
                   idx_hbm, tabT, tail_hbm,
                   out0T, out1T,
                   idx_v, mi_v, mj_v, bi_v, bj_v, slab_v, tail_v,
                   zflat_v, eT_v, rowflat_v, idxflat_v,
                   sp_acc):
    c = lax.axis_index("c")
    s = lax.axis_index("s")
    wid = s * _NC + c
    iota = lax.iota(jnp.int32, 16)

    def zrow(r, _):
        zflat_v[pl.ds(r * 16, 16)] = jnp.zeros((16,), jnp.int32)
        return 0
    lax.fori_loop(0, (_P2 * NF // 2) // 16, zrow, 0)
    pltpu.sync_copy(zflat_v,
                    sp_acc.at[pl.ds(s * (_P2 * NF // 2), _P2 * NF // 2)])
    plsc.subcore_barrier()

    full_tiles = nfull // 128
    base_t, extra_t = full_tiles // _NW, full_tiles % _NW
    lo = (wid * base_t + jnp.minimum(wid, extra_t)) * 128
    cnt = jnp.where(wid < extra_t, base_t + 1, base_t) * 128
    hi = lo + cnt
    hi_scan = jnp.where(wid == _NW - 1, jnp.int32(n_total), hi)
    @pl.when(wid == _NW - 1)
    def _():
        pltpu.sync_copy(tabT.at[:, pl.ds(jnp.int32(nfull), ntail)], tail_v)

    def scan_chunk(ic, n):
        pltpu.sync_copy(idx_hbm.at[pl.ds(ic * _IDXCH, _IDXCH)], idx_v)
        def scan_g(g, n):
            idx16 = idx_v[pl.ds(g * 16, 16)]
            m = jnp.logical_and(idx16 >= lo, idx16 < hi_scan)
            plsc.store_compressed(mi_v.at[pl.ds(n, 16)], idx16, mask=m)
            j16 = iota + (ic * _IDXCH + g * 16)
            plsc.store_compressed(mj_v.at[pl.ds(n, 16)], j16, mask=m)
            return n + jnp.max(plsc.all_reduce_population_count(m))
        return lax.fori_loop(0, _IDXCH // 16, scan_g, n)
    n = lax.fori_loop(0, B // _IDXCH, scan_chunk, jnp.int32(0))
    ng = (n + 15) // 16

    def extract_bin(slab_ref, nb, clampst, maxrel):
        def grp(g, _):
            k = g * 16
            rm = bi_v[pl.ds(k, 16)]
            jm = bj_v[pl.ds(k, 16)]
            valid = (iota + k) < nb
            rel = jnp.clip(rm - clampst, 0, jnp.int32(maxrel - 1))
            jbase = jnp.where(valid, jm * (NF // 2),
                              jnp.int32(B * NF // 2) + s)
            for fp in range(NF // 2):
                v0 = plsc.load_gather(slab_ref, [_i16(2 * fp), rel])
                v1 = plsc.load_gather(slab_ref, [_i16(2 * fp + 1), rel])
                w = plsc.pack(v0, v1, format=plsc.PackFormat.INTERLEAVED)
                wi = plsc.bitcast(w, jnp.int32)
                plsc.store_scatter(rowflat_v, [iota * (NF // 2) + fp], wi)
                plsc.store_scatter(idxflat_v, [iota * (NF // 2) + fp],
                                   jbase + fp)
            pltpu.sync_copy(rowflat_v, sp_acc.at[idxflat_v])
            return 0
        lax.fori_loop(0, (nb + 15) // 16, grp, 0)

    def compact_chunk(cov_lo, cov_hi):
        def cg(g, nb):
            rm = mi_v[pl.ds(g * 16, 16)]
            jm = mj_v[pl.ds(g * 16, 16)]
            valid = (iota + g * 16) < n
            m = valid & (rm >= cov_lo) & (rm < cov_hi)
            nbs = jnp.minimum(nb, jnp.int32(_BINCAP - 16))
            plsc.store_compressed(bi_v.at[pl.ds(nbs, 16)], rm, mask=m)
            plsc.store_compressed(bj_v.at[pl.ds(nbs, 16)], jm, mask=m)
            return nbs + jnp.max(plsc.all_reduce_population_count(m))
        return lax.fori_loop(0, ng, cg, jnp.int32(0))

    for ch in range(nchunk):
        cov_lo = lo + ch * _SLAB
        cov_hi = jnp.minimum(cov_lo + _SLAB, hi_scan)
        @pl.when(cov_lo < hi_scan)
        def _():
            clampst = jnp.minimum(cov_lo, jnp.int32(nfull - _SLAB))
            pltpu.sync_copy(tabT.at[:, pl.ds(clampst, _SLAB)], slab_v)
            nb = compact_chunk(jnp.maximum(cov_lo, clampst),
                               jnp.minimum(cov_hi, clampst + _SLAB))
            extract_bin(slab_v, nb, clampst, _SLAB)

    @pl.when(wid == _NW - 1)
    def _():
        nb = compact_chunk(jnp.int32(nfull), jnp.int32(n_total))
        extract_bin(tail_v, nb, jnp.int32(nfull), ntail)

    plsc.subcore_barrier()

    pltpu.sync_copy(sp_acc.at[pl.ds(s * (_P2 * NF // 2), _P2 * NF // 2)],
                    zflat_v)
    def tg(g, _):
        j0 = g * 16
        base = (iota + j0) * (NF // 2)
        for fp in range(NF // 2):
            w = plsc.load_gather(zflat_v, [base + fp])
            wb = plsc.bitcast(w, jnp.bfloat16)
            a, b = plsc.unpack(wb, format=plsc.PackFormat.INTERLEAVED)
            eT_v[2 * fp, pl.ds(j0, 16)] = a
            eT_v[2 * fp + 1, pl.ds(j0, 16)] = b
        return 0
    lax.fori_loop(0, _P2 // 16, tg, 0)
    @pl.when(c == 0)
    def _():
        pltpu.sync_copy(eT_v, out0T.at[:, pl.ds(s * _P2, _P2)])
    @pl.when(c == 1)
    def _():
        pltpu.sync_copy(eT_v, out1T.at[:, pl.ds(s * _P2, _P2)])


@functools.cache
def _sc_table_kernel(ntail, nfull, n_total, nchunk):
    f32, i32 = jnp.float32, jnp.int32
    outs = [
        jax.ShapeDtypeStruct((NF, B), f32),
        jax.ShapeDtypeStruct((NF, B), f32),
    ]
    kern = functools.partial(
        pl.kernel,
        mesh=plsc.VectorSubcoreMesh(core_axis_name="c", subcore_axis_name="s"),
        compiler_params=pltpu.CompilerParams(needs_layout_passes=False),
        out_type=outs,
        scratch_types=[
            pltpu.VMEM((_IDXCH,), i32),
            pltpu.VMEM((B,), i32),
            pltpu.VMEM((B,), i32),
            pltpu.VMEM((_BINCAP,), i32),
            pltpu.VMEM((_BINCAP,), i32),
            pltpu.VMEM((NF, _SLAB), f32),
            pltpu.VMEM((NF, ntail), f32),
            pltpu.VMEM((_P2 * NF // 2,), i32),
            pltpu.VMEM((NF, _P2), f32),
            pltpu.VMEM((128,), i32),
            pltpu.VMEM((128,), i32),
            pltpu.VMEM_SHARED((B * NF // 2 + 16,), i32),
        ],
    )

    def body(idx_hbm, tabT, *rest):
        return _sc_table_body(ntail, nfull, n_total, nchunk,
                              idx_hbm, tabT, None, *rest)

    return kern(body)


def _sc_bias_body(ci_hbm, ri_hbm, cb_hbm, rb_hbm, bs_out,
                  bidx_v, cb_v, rb_v, bsem):
    c = lax.axis_index("c")
    s = lax.axis_index("s")
    wid = s * _NC + c
    b0 = wid * 512
    pltpu.sync_copy(ci_hbm.at[pl.ds(b0, 512)], bidx_v)
    pltpu.async_copy(cb_hbm.at[bidx_v], cb_v, bsem).wait()
    pltpu.sync_copy(ri_hbm.at[pl.ds(b0, 512)], bidx_v)
    pltpu.async_copy(rb_hbm.at[bidx_v], rb_v, bsem).wait()
    def bsum(g, _):
        sl = pl.ds(g * 16, 16)
        cb_v[sl] = cb_v[sl] + rb_v[sl]
        return 0
    lax.fori_loop(0, 512 // 16, bsum, 0, unroll=4)
    pltpu.sync_copy(cb_v, bs_out.at[pl.ds(b0, 512)])


@functools.cache
def _sc_bias_kernel():
    f32, i32 = jnp.float32, jnp.int32
    return functools.partial(
        pl.kernel,
        mesh=plsc.VectorSubcoreMesh(core_axis_name="c", subcore_axis_name="s"),
        compiler_params=pltpu.CompilerParams(needs_layout_passes=False),
        out_type=[jax.ShapeDtypeStruct((B,), f32)],
        scratch_types=[
            pltpu.VMEM((512,), i32),
            pltpu.VMEM((512,), f32),
            pltpu.VMEM((512,), f32),
            pltpu.SemaphoreType.DMA,
        ],
    )(_sc_bias_body)


_BLK = 2048


def _tc_mlp_body(cfT, rfT, ce0, ce1, re0, re1, bs,
                 w1aT, w1bT, b1, w2T, b2,
                 ew1hT, ew1fT, eb1, ew2T, eb2, woT, bo, out):
    hT = jnp.maximum(
        jnp.dot(w1aT[...], cfT[...], preferred_element_type=jnp.float32)
        + jnp.dot(w1bT[...], rfT[...], preferred_element_type=jnp.float32)
        + b1[...], 0.0)
    h2T = jnp.maximum(
        jnp.dot(w2T[...], hT, preferred_element_type=jnp.float32) + b2[...],
        0.0)
    fmT = (ce0[...] + ce1[...]) * (re0[...] + re1[...])
    e1T = jnp.maximum(
        jnp.dot(ew1hT[...], h2T, preferred_element_type=jnp.float32)
        + jnp.dot(ew1fT[...], fmT, preferred_element_type=jnp.float32)
        + eb1[...], 0.0)
    e2T = jnp.maximum(
        jnp.dot(ew2T[...], e1T, preferred_element_type=jnp.float32)
        + eb2[...], 0.0)
    eoT = jnp.dot(woT[...], e2T, preferred_element_type=jnp.float32) + bo[...]
    out[...] = bs[...] + eoT


def _full(shape):
    return pl.BlockSpec(shape, lambda i: (0, 0))


def _tc_mlp(cfT, rfT, ce0, ce1, re0, re1, bs,
            w1aT, w1bT, b1, w2T, b2,
            ew1hT, ew1fT, eb1, ew2T, eb2, woT, bo, interpret=False):
    grid = (B // _BLK,)
    colblk = lambda h: pl.BlockSpec((h, _BLK), lambda i: (0, i))
    return pl.pallas_call(
        _tc_mlp_body,
        grid=grid,
        in_specs=[
            colblk(64), colblk(64),
            colblk(NF), colblk(NF), colblk(NF), colblk(NF), colblk(1),
            _full((16, 64)), _full((16, 64)), _full((16, 1)),
            _full((8, 16)), _full((8, 1)),
            _full((8, 8)), _full((8, NF)), _full((8, 1)),
            _full((4, 8)), _full((4, 1)),
            _full((1, 4)), _full((1, 1)),
        ],
        out_specs=pl.BlockSpec((1, _BLK), lambda i: (0, i)),
        out_shape=jax.ShapeDtypeStruct((1, B), jnp.float32),
        interpret=interpret,
    )(cfT, rfT, ce0, ce1, re0, re1, bs,
      w1aT, w1bT, b1, w2T, b2,
      ew1hT, ew1fT, eb1, ew2T, eb2, woT, bo)


def kernel(coil_indices, recipe_indices, coil_features, recipe_features,
           coil_emb, recipe_emb, coil_bias, recipe_bias,
           mlp_W1, mlp_b1, mlp_W2, mlp_b2,
           ens_W1, ens_b1, ens_W2, ens_b2, ens_Wo, ens_bo):
    ci = coil_indices.astype(jnp.int32)
    ri = recipe_indices.astype(jnp.int32)
    ce0, ce1 = _sc_table_kernel(NCOIL - _COIL_FULL, _COIL_FULL,
                                NCOIL, 12)(
        ci, coil_emb.T)
    re0, re1 = _sc_table_kernel(NREC - _REC_FULL, _REC_FULL,
                                NREC, 2)(
        ri, recipe_emb.T)
    (bs,) = _sc_bias_kernel()(
        ci, ri, coil_bias.reshape(-1), recipe_bias.reshape(-1))
    w1T = mlp_W1.T
    pred = _tc_mlp(
        coil_features.T, recipe_features.T, ce0, ce1, re0, re1,
        bs.reshape(1, B),
        w1T[:, :64], w1T[:, 64:], mlp_b1.reshape(16, 1),
        mlp_W2.T, mlp_b2.reshape(8, 1),
        ens_W1.T[:, :8], ens_W1.T[:, 8:], ens_b1.reshape(8, 1),
        ens_W2.T, ens_b2.reshape(4, 1),
        ens_Wo.T, ens_bo.reshape(1, 1))
    return pred[0]
